# baseline (speedup 1.0000x reference)
import functools
import math

import jax
import jax.numpy as jnp
from jax.experimental import pallas as pl
from jax.experimental.pallas import tpu as pltpu

_EPS = 1.1920929e-07


def _routing_body(n_tokens, t_a, nsteps, x_hbm, wt_ref, meta_ref, aux_ref,
                  xb, xsem, c1, c2, ms):
    blk = pl.program_id(0)

    @pl.when(blk == 0)
    def _init():
        c1[...] = jnp.zeros_like(c1)
        c2[...] = jnp.zeros_like(c2)
        ms[...] = jnp.zeros_like(ms)
        pltpu.make_async_copy(
            x_hbm.at[pl.ds(0, t_a)], xb.at[0], xsem.at[0]).start()

    @pl.when(blk + 1 < nsteps)
    def _prefetch():
        nxt = jax.lax.rem(blk + 1, 2)
        pltpu.make_async_copy(
            x_hbm.at[pl.ds((blk + 1) * t_a, t_a)], xb.at[nxt],
            xsem.at[nxt]).start()

    cur = jax.lax.rem(blk, 2)
    pltpu.make_async_copy(
        x_hbm.at[pl.ds(blk * t_a, t_a)], xb.at[cur], xsem.at[cur]).wait()
    x = xb[cur]
    logits = jax.lax.dot_general(
        x, wt_ref[...], (((1,), (0,)), ((), ())),
        preferred_element_type=jnp.float32)
    t, e = logits.shape

    m1 = jnp.max(logits, axis=1, keepdims=True)
    ex = jnp.exp(logits - m1)
    gates = ex / jnp.sum(ex, axis=1, keepdims=True)

    lane = jax.lax.broadcasted_iota(jnp.int32, (t, e), 1)
    i1 = jnp.min(jnp.where(logits == m1, lane, jnp.int32(10**9)),
                 axis=1, keepdims=True)
    mask1 = (lane == i1).astype(jnp.float32)
    masked = jnp.where(lane == i1, -jnp.inf, logits)
    m2 = jnp.max(masked, axis=1, keepdims=True)
    i2 = jnp.min(jnp.where(masked == m2, lane, jnp.int32(10**9)),
                 axis=1, keepdims=True)
    mask2 = (lane == i2).astype(jnp.float32)

    r = jax.lax.broadcasted_iota(jnp.int32, (t, t), 0)
    c = jax.lax.broadcasted_iota(jnp.int32, (t, t), 1)
    tri = (r >= c).astype(jnp.float32)
    cs1 = jax.lax.dot_general(tri, mask1, (((1,), (0,)), ((), ())),
                              preferred_element_type=jnp.float32)
    cs2 = jax.lax.dot_general(tri, mask2, (((1,), (0,)), ((), ())),
                              preferred_element_type=jnp.float32)
    loc1 = cs1 - 1.0 + c1[...]
    loc2 = cs2 - 1.0 + c2[...]

    meta_ref[...] = jnp.concatenate(
        [gates, mask1, mask2, loc1 * mask1, loc2 * mask2], axis=1)

    c1[...] += jnp.sum(mask1, axis=0, keepdims=True)
    c2[...] += jnp.sum(mask2, axis=0, keepdims=True)
    ms[...] += jnp.sum(gates, axis=0, keepdims=True)
    inv_n = 1.0 / n_tokens
    laux = jnp.sum((ms[...] * inv_n) * (c1[...] * inv_n),
                   axis=1, keepdims=True) * e
    aux_ref[...] = jnp.concatenate(
        [c1[...], jnp.broadcast_to(laux, (1, e))], axis=1)


def _expand_compute(capacity, e, cnt1, meta):
    g = meta[:, :e]
    s1 = meta[:, e:2 * e]
    s2 = meta[:, 2 * e:3 * e]
    l1 = meta[:, 3 * e:4 * e]
    l2 = meta[:, 4 * e:5 * e] + cnt1 * s2

    m1c = jnp.where(l1 < capacity, s1, 0.0)
    m2c = jnp.where(l2 < capacity, s2, 0.0)
    g1 = jnp.sum(g * m1c, axis=1, keepdims=True)
    g2 = jnp.sum(g * m2c, axis=1, keepdims=True)
    denom = jnp.maximum(g1 + g2, _EPS)
    w = (g1 / denom) * m1c + (g2 / denom) * m2c
    p = jnp.where(m1c > 0, l1, jnp.where(m2c > 0, l2, -1.0)).astype(jnp.int32)

    t = meta.shape[0]
    ci = jax.lax.broadcasted_iota(jnp.int32, (t, e, capacity), 2)
    hit = ci == p[:, :, None]
    cw = jnp.where(hit, w[:, :, None], 0.0).reshape(t * e, capacity)
    dm = hit & (w[:, :, None] > 0.0)
    return cw, dm


def _expand_body(capacity, t_b, nsteps, aux_ref, meta_ref, cw_hbm, dm_ref,
                 cwb, sems):
    i = pl.program_id(0)
    e = aux_ref.shape[1] // 2
    cnt1 = aux_ref[:, :e]
    meta = meta_ref[pl.ds(i * t_b, t_b), :]
    cw_val, dm_val = _expand_compute(capacity, e, cnt1, meta)
    dm_ref[...] = dm_val

    te = t_b * e
    slot = jax.lax.rem(i, 2)
    for s in (0, 1):
        @pl.when(slot == s)
        def _(s=s):
            @pl.when(i >= 2)
            def _():
                pltpu.make_async_copy(
                    cwb.at[s], cw_hbm.at[pl.ds((i - 2) * te, te)],
                    sems.at[s]).wait()
            cwb[s] = cw_val
            pltpu.make_async_copy(
                cwb.at[s], cw_hbm.at[pl.ds(i * te, te)], sems.at[s]).start()

    @pl.when(i == nsteps - 1)
    def _drain():
        for s2 in (0, 1):
            blk = jnp.where(jax.lax.rem(jnp.int32(nsteps - 1), 2) == s2,
                            nsteps - 1, nsteps - 2)
            pltpu.make_async_copy(
                cwb.at[s2], cw_hbm.at[pl.ds(blk * te, te)], sems.at[s2]).wait()


@jax.jit
def kernel(x, W):
    n, d = x.shape
    e = W.shape[0]
    capacity = 2 * math.ceil(n / e)
    t_a = min(512, n)
    nb_a = n // t_a
    t_b = min(128, n)
    nb_b = n // t_b

    wt = W.T

    meta, aux = pl.pallas_call(
        functools.partial(_routing_body, n, t_a, nb_a),
        grid=(nb_a,),
        in_specs=[
            pl.BlockSpec(memory_space=pl.ANY),
            pl.BlockSpec((d, e), lambda i: (0, 0)),
        ],
        out_specs=[
            pl.BlockSpec((t_a, 5 * e), lambda i: (i, 0)),
            pl.BlockSpec((1, 2 * e), lambda i: (0, 0)),
        ],
        out_shape=[
            jax.ShapeDtypeStruct((n, 5 * e), jnp.float32),
            jax.ShapeDtypeStruct((1, 2 * e), jnp.float32),
        ],
        scratch_shapes=[
            pltpu.VMEM((2, t_a, d), jnp.float32),
            pltpu.SemaphoreType.DMA((2,)),
        ] + [pltpu.VMEM((1, e), jnp.float32)] * 3,
        compiler_params=pltpu.CompilerParams(
            dimension_semantics=("arbitrary",)),
    )(x, wt)

    cw2, dm2 = pl.pallas_call(
        functools.partial(_expand_body, capacity, t_b, nb_b),
        grid=(nb_b,),
        in_specs=[
            pl.BlockSpec((1, 2 * e), lambda i: (0, 0)),
            pl.BlockSpec((n, 5 * e), lambda i: (0, 0)),
        ],
        out_specs=[
            pl.BlockSpec(memory_space=pl.ANY),
            pl.BlockSpec((t_b, e, capacity), lambda i: (i, 0, 0)),
        ],
        out_shape=[
            jax.ShapeDtypeStruct((n * e, capacity), jnp.float32),
            jax.ShapeDtypeStruct((n, e, capacity), jnp.bool_),
        ],
        scratch_shapes=[
            pltpu.VMEM((2, t_b * e, capacity), jnp.float32),
            pltpu.SemaphoreType.DMA((2,)),
        ],
        compiler_params=pltpu.CompilerParams(
            dimension_semantics=("arbitrary",)),
    )(aux, meta)

    laux = aux[0, e]
    return (laux, cw2.reshape(n, e, capacity), dm2)


# speedup vs baseline: 1.2281x; 1.2281x over previous
import functools
import math

import jax
import jax.numpy as jnp
from jax.experimental import pallas as pl
from jax.experimental.pallas import tpu as pltpu

_EPS = 1.1920929e-07


def _routing_body(n_tokens, x_ref, wt_ref, meta_ref, aux_ref, c1, c2, ms):
    blk = pl.program_id(0)

    @pl.when(blk == 0)
    def _init():
        c1[...] = jnp.zeros_like(c1)
        c2[...] = jnp.zeros_like(c2)
        ms[...] = jnp.zeros_like(ms)

    x = x_ref[...]
    logits = jax.lax.dot_general(
        x, wt_ref[...], (((1,), (0,)), ((), ())),
        preferred_element_type=jnp.float32)
    t, e = logits.shape

    m1 = jnp.max(logits, axis=1, keepdims=True)
    ex = jnp.exp(logits - m1)
    gates = ex / jnp.sum(ex, axis=1, keepdims=True)

    lane = jax.lax.broadcasted_iota(jnp.int32, (t, e), 1)
    i1 = jnp.min(jnp.where(logits == m1, lane, jnp.int32(10**9)),
                 axis=1, keepdims=True)
    mask1 = (lane == i1).astype(jnp.float32)
    masked = jnp.where(lane == i1, -jnp.inf, logits)
    m2 = jnp.max(masked, axis=1, keepdims=True)
    i2 = jnp.min(jnp.where(masked == m2, lane, jnp.int32(10**9)),
                 axis=1, keepdims=True)
    mask2 = (lane == i2).astype(jnp.float32)

    r = jax.lax.broadcasted_iota(jnp.int32, (t, t), 0)
    c = jax.lax.broadcasted_iota(jnp.int32, (t, t), 1)
    tri = (r >= c).astype(jnp.float32)
    cs1 = jax.lax.dot_general(tri, mask1, (((1,), (0,)), ((), ())),
                              preferred_element_type=jnp.float32)
    cs2 = jax.lax.dot_general(tri, mask2, (((1,), (0,)), ((), ())),
                              preferred_element_type=jnp.float32)
    loc1 = cs1 - 1.0 + c1[...]
    loc2 = cs2 - 1.0 + c2[...]

    meta_ref[...] = jnp.concatenate(
        [gates, mask1, mask2, loc1 * mask1, loc2 * mask2], axis=1)

    c1[...] += jnp.sum(mask1, axis=0, keepdims=True)
    c2[...] += jnp.sum(mask2, axis=0, keepdims=True)
    ms[...] += jnp.sum(gates, axis=0, keepdims=True)
    inv_n = 1.0 / n_tokens
    laux = jnp.sum((ms[...] * inv_n) * (c1[...] * inv_n),
                   axis=1, keepdims=True) * e
    aux_ref[...] = jnp.concatenate(
        [c1[...], jnp.broadcast_to(laux, (1, e))], axis=1)


def _expand_compute(capacity, e, cnt1, meta):
    g = meta[:, :e]
    s1 = meta[:, e:2 * e]
    s2 = meta[:, 2 * e:3 * e]
    l1 = meta[:, 3 * e:4 * e]
    l2 = meta[:, 4 * e:5 * e] + cnt1 * s2

    m1c = jnp.where(l1 < capacity, s1, 0.0)
    m2c = jnp.where(l2 < capacity, s2, 0.0)
    g1 = jnp.sum(g * m1c, axis=1, keepdims=True)
    g2 = jnp.sum(g * m2c, axis=1, keepdims=True)
    denom = jnp.maximum(g1 + g2, _EPS)
    w = (g1 / denom) * m1c + (g2 / denom) * m2c
    p = jnp.where(m1c > 0, l1, jnp.where(m2c > 0, l2, -1.0)).astype(jnp.int32)

    t = meta.shape[0]
    ci = jax.lax.broadcasted_iota(jnp.int32, (t, e, capacity), 2)
    hit = ci == p[:, :, None]
    cw = jnp.where(hit, w[:, :, None], 0.0).reshape(t * e, capacity)
    dm = (hit & (w[:, :, None] > 0.0)).astype(jnp.int8)
    return cw, dm


def _expand_body(capacity, t_b, nsteps, aux_ref, meta_ref, cw_hbm, dm_hbm,
                 cwb, dmb, sems):
    i = pl.program_id(0)
    e = aux_ref.shape[1] // 2
    cnt1 = aux_ref[:, :e]
    meta = meta_ref[pl.ds(i * t_b, t_b), :]
    cw_val, dm_val = _expand_compute(capacity, e, cnt1, meta)

    te = t_b * e
    slot = jax.lax.rem(i, 2)
    for s in (0, 1):
        @pl.when(slot == s)
        def _(s=s):
            @pl.when(i >= 2)
            def _():
                pltpu.make_async_copy(
                    cwb.at[s], cw_hbm.at[pl.ds((i - 2) * te, te)],
                    sems.at[0, s]).wait()
                pltpu.make_async_copy(
                    dmb.at[s], dm_hbm.at[pl.ds((i - 2) * t_b, t_b)],
                    sems.at[1, s]).wait()
            cwb[s] = cw_val
            dmb[s] = dm_val
            pltpu.make_async_copy(
                cwb.at[s], cw_hbm.at[pl.ds(i * te, te)], sems.at[0, s]).start()
            pltpu.make_async_copy(
                dmb.at[s], dm_hbm.at[pl.ds(i * t_b, t_b)], sems.at[1, s]).start()

    @pl.when(i == nsteps - 1)
    def _drain():
        for s2 in (0, 1):
            blk = jnp.where(jax.lax.rem(jnp.int32(nsteps - 1), 2) == s2,
                            nsteps - 1, nsteps - 2)
            pltpu.make_async_copy(
                cwb.at[s2], cw_hbm.at[pl.ds(blk * te, te)], sems.at[0, s2]).wait()
            pltpu.make_async_copy(
                dmb.at[s2], dm_hbm.at[pl.ds(blk * t_b, t_b)], sems.at[1, s2]).wait()


@jax.jit
def kernel(x, W):
    n, d = x.shape
    e = W.shape[0]
    capacity = 2 * math.ceil(n / e)
    t_a = min(512, n)
    nb_a = n // t_a
    t_b = min(128, n)
    nb_b = n // t_b

    wt = W.T

    meta, aux = pl.pallas_call(
        functools.partial(_routing_body, n),
        grid=(nb_a,),
        in_specs=[
            pl.BlockSpec((t_a, d), lambda i: (i, 0)),
            pl.BlockSpec((d, e), lambda i: (0, 0)),
        ],
        out_specs=[
            pl.BlockSpec((t_a, 5 * e), lambda i: (i, 0)),
            pl.BlockSpec((1, 2 * e), lambda i: (0, 0)),
        ],
        out_shape=[
            jax.ShapeDtypeStruct((n, 5 * e), jnp.float32),
            jax.ShapeDtypeStruct((1, 2 * e), jnp.float32),
        ],
        scratch_shapes=[pltpu.VMEM((1, e), jnp.float32)] * 3,
        compiler_params=pltpu.CompilerParams(
            dimension_semantics=("arbitrary",)),
    )(x, wt)

    cw2, dm2 = pl.pallas_call(
        functools.partial(_expand_body, capacity, t_b, nb_b),
        grid=(nb_b,),
        in_specs=[
            pl.BlockSpec((1, 2 * e), lambda i: (0, 0)),
            pl.BlockSpec((n, 5 * e), lambda i: (0, 0)),
        ],
        out_specs=[
            pl.BlockSpec(memory_space=pl.ANY),
            pl.BlockSpec(memory_space=pl.ANY),
        ],
        out_shape=[
            jax.ShapeDtypeStruct((n * e, capacity), jnp.float32),
            jax.ShapeDtypeStruct((n, e, capacity), jnp.int8),
        ],
        scratch_shapes=[
            pltpu.VMEM((2, t_b * e, capacity), jnp.float32),
            pltpu.VMEM((2, t_b, e, capacity), jnp.int8),
            pltpu.SemaphoreType.DMA((2, 2)),
        ],
        compiler_params=pltpu.CompilerParams(
            dimension_semantics=("arbitrary",)),
    )(aux, meta)

    laux = aux[0, e]
    return (laux, cw2.reshape(n, e, capacity), dm2.view(jnp.bool_))
